# Initial kernel scaffold; baseline (speedup 1.0000x reference)
import jax
import jax.numpy as jnp
from jax.experimental import pallas as pl


def kernel(map_token, actor_token, light_token, map_pe_w, actor_pe_w, light_pe_w, time_pe_w, pos_enc):
    raise NotImplementedError("write your pallas kernel here")


# speedup vs baseline: 1.4291x; 1.4291x over previous
import jax
import jax.numpy as jnp
from jax.experimental import pallas as pl


def _qpe_body(map_t, actor_t, light_t, map_pe, actor_pe, light_pe, time_pe,
              pos, map_o, actor_o, light_o):
    S = map_t.shape[1]
    T = actor_t.shape[1]
    N = actor_t.shape[2]
    L = light_t.shape[2]
    D = map_t.shape[-1]

    pos_all = pos[...]
    map_comb = map_pe[...] + pos_all[:S]
    map_o[...] = map_t[...] + map_comb[None]

    time_comb = (time_pe[:T] + pos_all[:T]).reshape(1, T, 1, D)
    actor_comb = (actor_pe[:N] + pos_all[:N]).reshape(1, 1, N, D)
    actor_o[...] = actor_t[...] + actor_comb + time_comb

    light_comb = (light_pe[:L] + pos_all[:L]).reshape(1, 1, L, D)
    light_o[...] = light_t[...] + light_comb + time_comb


def kernel(map_token, actor_token, light_token, map_pe_w, actor_pe_w,
           light_pe_w, time_pe_w, pos_enc):
    B, S, D = map_token.shape
    _, T, N, _ = actor_token.shape
    L = light_token.shape[2]

    whole = lambda shape: pl.BlockSpec(shape, lambda b: (0,) * len(shape))
    outs = pl.pallas_call(
        _qpe_body,
        grid=(B,),
        in_specs=[
            pl.BlockSpec((1, S, D), lambda b: (b, 0, 0)),
            pl.BlockSpec((1, T, N, D), lambda b: (b, 0, 0, 0)),
            pl.BlockSpec((1, T, L, D), lambda b: (b, 0, 0, 0)),
            whole(map_pe_w.shape),
            whole(actor_pe_w.shape),
            whole(light_pe_w.shape),
            whole(time_pe_w.shape),
            whole(pos_enc.shape),
        ],
        out_specs=[
            pl.BlockSpec((1, S, D), lambda b: (b, 0, 0)),
            pl.BlockSpec((1, T, N, D), lambda b: (b, 0, 0, 0)),
            pl.BlockSpec((1, T, L, D), lambda b: (b, 0, 0, 0)),
        ],
        out_shape=[
            jax.ShapeDtypeStruct((B, S, D), map_token.dtype),
            jax.ShapeDtypeStruct((B, T, N, D), actor_token.dtype),
            jax.ShapeDtypeStruct((B, T, L, D), light_token.dtype),
        ],
    )(map_token, actor_token, light_token, map_pe_w, actor_pe_w,
      light_pe_w, time_pe_w, pos_enc)
    return tuple(outs)
